# baseline (speedup 1.0000x reference)
import functools

import jax
import jax.numpy as jnp
from jax import lax
from jax.experimental import pallas as pl
from jax.experimental.pallas import tpu as pltpu
from jax.experimental.pallas import tpu_sc as plsc

NC = 2
NS = 16
NW = NC * NS
BBLK = 128
VPAD = 128
TBLK = 8192
BT = 2048


def _transpose_pad_table(ttT):
    D, V = ttT.shape

    def body(in_ref, out_ref):
        out_ref[:, 0:D] = in_ref[...].T

    return pl.pallas_call(
        body,
        grid=(pl.cdiv(V, TBLK),),
        in_specs=[pl.BlockSpec((D, TBLK), lambda i: (0, i))],
        out_specs=pl.BlockSpec((TBLK, VPAD), lambda i: (i, 0)),
        out_shape=jax.ShapeDtypeStruct((V, VPAD), jnp.float32),
    )(ttT)


def _make_gather(T, B, D, T0, T1):
    mesh = plsc.VectorSubcoreMesh(core_axis_name="c", subcore_axis_name="s")
    PCOL = 256
    LANES = 16

    @functools.partial(
        pl.kernel,
        mesh=mesh,
        compiler_params=pltpu.CompilerParams(
            use_tc_tiling_on_sc=True, needs_layout_passes=False),
        out_type=jax.ShapeDtypeStruct((T, B, VPAD), jnp.float32),
        scratch_types=[
            pltpu.VMEM((T1 - T0, BBLK), jnp.int32),
            pltpu.VMEM((D, PCOL), jnp.float32),
            [pltpu.VMEM((BBLK, VPAD), jnp.float32)] * 2,
            [pltpu.SemaphoreType.DMA] * 2,
            [pltpu.SemaphoreType.DMA] * 2,
            pltpu.SemaphoreType.DMA,
        ],
    )
    def gat(xT_hbm, tt_hbm, posT_hbm, g_hbm, idx_v, pos_v, gbufs, gsems,
            osems, sem0):
        wid = lax.axis_index("s") * NC + lax.axis_index("c")
        bcol = pl.multiple_of(wid * BBLK, BBLK)

        cp_pos = pltpu.async_copy(posT_hbm.at[:, pl.ds(0, PCOL)], pos_v, sem0)
        cp_idx = pltpu.async_copy(
            xT_hbm.at[pl.ds(T0, T1 - T0), pl.ds(bcol, BBLK)], idx_v, sem0)
        cp_pos.wait()
        cp_idx.wait()

        iota = lax.iota(jnp.int32, LANES)
        fgrp = [iota + g * LANES for g in range(D // LANES)]

        def issue_gather(i, j):
            pltpu.async_copy(tt_hbm.at[idx_v.at[i]], gbufs[j], gsems[j])

        def wait_gather(i, j):
            pltpu.make_async_copy(
                tt_hbm.at[idx_v.at[i]], gbufs[j], gsems[j]).wait()

        def issue_out(i, j):
            pltpu.async_copy(
                gbufs[j], g_hbm.at[T0 + i, pl.ds(bcol, BBLK)], osems[j])

        def wait_out(i, j):
            pltpu.make_async_copy(
                gbufs[j], g_hbm.at[T0 + i, pl.ds(bcol, BBLK)],
                osems[j]).wait()

        issue_gather(0, 0)
        issue_gather(1, 1)

        N = T1 - T0

        @pl.loop(0, N, step=2)
        def _(i0):
            for j in range(2):
                i = i0 + j
                wait_gather(i, j)

                @pl.when(i >= 2)
                def _():
                    wait_out(i - 2, j)

                tvec = jnp.full((LANES,), T0 + i, jnp.int32)
                pchunk = [plsc.load_gather(pos_v, [fgrp[c], tvec])
                          for c in range(D // LANES)]

                @pl.loop(0, BBLK)
                def _(r):
                    for c in range(D // LANES):
                        plsc.addupdate(
                            gbufs[j].at[r, pl.ds(c * LANES, LANES)],
                            pchunk[c])

                issue_out(i, j)

                @pl.when(i + 2 < N)
                def _():
                    issue_gather(i + 2, j)

        wait_out(N - 2, 0)
        wait_out(N - 1, 1)

    return gat


def _transpose_out(g, D, out_buf, T0, T1):
    T, B, _ = g.shape
    NT = T1 - T0

    def body(*refs):
        g_ref, out_ref = refs[0], refs[-1]
        i0 = lax.broadcasted_iota(jnp.int32, (D, D), 0)
        i1 = lax.broadcasted_iota(jnp.int32, (D, D), 1)
        ident = (i0 == i1).astype(jnp.float32)
        out_ref[0] = lax.dot_general(
            ident, g_ref[0][:, 0:D],
            dimension_numbers=(((1,), (1,)), ((), ())),
            preferred_element_type=jnp.float32,
            precision=lax.Precision.HIGHEST)

    args = (g,) if out_buf is None else (g, out_buf)
    in_specs = [
        pl.BlockSpec((1, BT, VPAD), lambda t, b: (T0 + t, b, 0)),
    ]
    aliases = {}
    if out_buf is not None:
        in_specs.append(pl.BlockSpec((1, D, BT), lambda t, b: (T0 + t, 0, b)))
        aliases = {1: 0}

    return pl.pallas_call(
        body,
        grid=(NT, B // BT),
        in_specs=in_specs,
        out_specs=pl.BlockSpec((1, D, BT), lambda t, b: (T0 + t, 0, b)),
        out_shape=jax.ShapeDtypeStruct((T, D, B), jnp.float32),
        input_output_aliases=aliases,
    )(*args)


@jax.jit
def kernel(x, token_table, pos_table):
    B, T = x.shape
    V, D = token_table.shape

    xT = x.T
    posT = pos_table.T
    ttp = _transpose_pad_table(token_table.T)

    TH = (T // 2) & ~7
    g1 = _make_gather(T, B, D, 0, TH)(xT, ttp, posT)
    g2 = _make_gather(T, B, D, TH, T)(xT, ttp, posT)
    out_buf = _transpose_out(g1, D, None, 0, TH)
    out_buf = _transpose_out(g2, D, out_buf, TH, T)
    return out_buf.transpose(2, 0, 1)


# speedup vs baseline: 1.1578x; 1.1578x over previous
import functools

import jax
import jax.numpy as jnp
from jax import lax
from jax.experimental import pallas as pl
from jax.experimental.pallas import tpu as pltpu
from jax.experimental.pallas import tpu_sc as plsc

NC = 2
NS = 16
NW = NC * NS
BBLK = 128
VPAD = 128
TBLK = 8192
BT = 2048


def _transpose_pad_table(ttT):
    D, V = ttT.shape

    def body(in_ref, out_ref):
        out_ref[:, 0:D] = in_ref[...].T

    return pl.pallas_call(
        body,
        grid=(pl.cdiv(V, TBLK),),
        in_specs=[pl.BlockSpec((D, TBLK), lambda i: (0, i))],
        out_specs=pl.BlockSpec((TBLK, VPAD), lambda i: (i, 0)),
        out_shape=jax.ShapeDtypeStruct((V, VPAD), jnp.float32),
    )(ttT)


def _make_gather(T, B, D, T0, T1):
    mesh = plsc.VectorSubcoreMesh(core_axis_name="c", subcore_axis_name="s")
    PCOL = 256
    LANES = 16

    @functools.partial(
        pl.kernel,
        mesh=mesh,
        compiler_params=pltpu.CompilerParams(
            use_tc_tiling_on_sc=True, needs_layout_passes=False),
        out_type=jax.ShapeDtypeStruct((T, B, VPAD), jnp.float32),
        scratch_types=[
            pltpu.VMEM((T1 - T0, BBLK), jnp.int32),
            pltpu.VMEM((D, PCOL), jnp.float32),
            [pltpu.VMEM((BBLK, VPAD), jnp.float32)] * 2,
            [pltpu.SemaphoreType.DMA] * 2,
            [pltpu.SemaphoreType.DMA] * 2,
            pltpu.SemaphoreType.DMA,
        ],
    )
    def gat(xT_hbm, tt_hbm, posT_hbm, g_hbm, idx_v, pos_v, gbufs, gsems,
            osems, sem0):
        wid = lax.axis_index("s") * NC + lax.axis_index("c")
        bcol = pl.multiple_of(wid * BBLK, BBLK)

        cp_pos = pltpu.async_copy(posT_hbm.at[:, pl.ds(0, PCOL)], pos_v, sem0)
        cp_idx = pltpu.async_copy(
            xT_hbm.at[pl.ds(T0, T1 - T0), pl.ds(bcol, BBLK)], idx_v, sem0)
        cp_pos.wait()
        cp_idx.wait()

        iota = lax.iota(jnp.int32, LANES)
        fgrp = [iota + g * LANES for g in range(D // LANES)]

        def issue_gather(i, j):
            pltpu.async_copy(tt_hbm.at[idx_v.at[i]], gbufs[j], gsems[j])

        def wait_gather(i, j):
            pltpu.make_async_copy(
                tt_hbm.at[idx_v.at[i]], gbufs[j], gsems[j]).wait()

        def issue_out(i, j):
            pltpu.async_copy(
                gbufs[j], g_hbm.at[T0 + i, pl.ds(bcol, BBLK)], osems[j])

        def wait_out(i, j):
            pltpu.make_async_copy(
                gbufs[j], g_hbm.at[T0 + i, pl.ds(bcol, BBLK)],
                osems[j]).wait()

        issue_gather(0, 0)
        issue_gather(1, 1)

        N = T1 - T0

        @pl.loop(0, N, step=2)
        def _(i0):
            for j in range(2):
                i = i0 + j
                wait_gather(i, j)

                @pl.when(i >= 2)
                def _():
                    wait_out(i - 2, j)

                tvec = jnp.full((LANES,), T0 + i, jnp.int32)
                pchunk = [plsc.load_gather(pos_v, [fgrp[c], tvec])
                          for c in range(D // LANES)]

                @pl.loop(0, BBLK)
                def _(r):
                    for c in range(D // LANES):
                        plsc.addupdate(
                            gbufs[j].at[r, pl.ds(c * LANES, LANES)],
                            pchunk[c])

                issue_out(i, j)

                @pl.when(i + 2 < N)
                def _():
                    issue_gather(i + 2, j)

        wait_out(N - 2, 0)
        wait_out(N - 1, 1)

    return gat


def _transpose_out(g, D, out_buf, T0, T1):
    T, B, _ = g.shape
    NT = T1 - T0

    def body(*refs):
        g_ref, out_ref = refs[0], refs[-1]
        i0 = lax.broadcasted_iota(jnp.int32, (D, D), 0)
        i1 = lax.broadcasted_iota(jnp.int32, (D, D), 1)
        ident = (i0 == i1).astype(jnp.float32)
        out_ref[0] = lax.dot_general(
            ident, g_ref[0][:, 0:D],
            dimension_numbers=(((1,), (1,)), ((), ())),
            preferred_element_type=jnp.float32,
            precision=lax.Precision.DEFAULT)

    args = (g,) if out_buf is None else (g, out_buf)
    in_specs = [
        pl.BlockSpec((1, BT, VPAD), lambda t, b: (T0 + t, b, 0)),
    ]
    aliases = {}
    if out_buf is not None:
        in_specs.append(pl.BlockSpec((1, D, BT), lambda t, b: (T0 + t, 0, b)))
        aliases = {1: 0}

    return pl.pallas_call(
        body,
        grid=(NT, B // BT),
        in_specs=in_specs,
        out_specs=pl.BlockSpec((1, D, BT), lambda t, b: (T0 + t, 0, b)),
        out_shape=jax.ShapeDtypeStruct((T, D, B), jnp.float32),
        input_output_aliases=aliases,
    )(*args)


@jax.jit
def kernel(x, token_table, pos_table):
    B, T = x.shape
    V, D = token_table.shape

    xT = x.T
    posT = pos_table.T
    ttp = _transpose_pad_table(token_table.T)

    TH = (T // 2) & ~7
    g1 = _make_gather(T, B, D, 0, TH)(xT, ttp, posT)
    g2 = _make_gather(T, B, D, TH, T)(xT, ttp, posT)
    out_buf = _transpose_out(g1, D, None, 0, TH)
    out_buf = _transpose_out(g2, D, out_buf, TH, T)
    return out_buf.transpose(2, 0, 1)
